# baseline (speedup 1.0000x reference)
import jax
import jax.numpy as jnp
from jax.experimental import pallas as pl
from jax.experimental.pallas import tpu as pltpu

D_MODEL = 2048
D_SPACE = 64
GROUPS = (
    (0, 256, 8),
    (256, 256, 8),
    (512, 256, 8),
    (768, 256, 8),
    (1024, 256, 8),
    (1280, 1024, 4),
    (2304, 1024, 4),
)
EMB_ROWS = ((0, 256), (256, 512), (512, 768), (512, 768), (768, 1024),
            (1024, 2048), (2048, 3072))
N_OUT = 3328
N_PROJ = 448
TILE = 256
N_TILES = 16


def _router_kernel(x_ref, w_ref, b_ref, emb_ref, o_ref, w_s, e_s, l_s):
    i = pl.program_id(0)

    @pl.when(i == 0)
    def _prep():
        w_s[...] = w_ref[...].T.astype(jnp.bfloat16)
        emb = emb_ref[...]
        norm = jnp.maximum(
            jnp.sqrt(jnp.sum(emb * emb, axis=-1, keepdims=True)), 1e-12)
        emb_n = (emb / norm).astype(jnp.bfloat16)
        e_s[...] = jnp.zeros((N_PROJ, N_OUT), dtype=jnp.bfloat16)
        for g, (off, width, _) in enumerate(GROUPS):
            a, bb = EMB_ROWS[g]
            e_s[64 * g:64 * (g + 1), off:off + width] = emb_n[a:bb].T
        l_s[...] = jnp.zeros((2, TILE, N_OUT), dtype=jnp.float32)

    xb = x_ref[...].astype(jnp.bfloat16)
    h = jnp.dot(xb, w_s[...], preferred_element_type=jnp.float32)
    h = (h + b_ref[...]).astype(jnp.bfloat16)
    l_s[i % 2] = jnp.dot(h, e_s[...], preferred_element_type=jnp.float32)

    logits = l_s[(i + 1) % 2]
    for off, width, k in GROUPS:
        l = logits[:, off:off + width]
        m = jnp.max(l, axis=-1, keepdims=True)
        cur, mv = l, m
        for _ in range(k - 1):
            cur = jnp.where(cur >= mv, -jnp.inf, cur)
            mv = jnp.max(cur, axis=-1, keepdims=True)
        ex = jnp.exp(l - m)
        mex = jnp.where(l >= mv, ex, 0.0)
        se = jnp.sum(mex, axis=-1, keepdims=True)
        o_ref[:, off:off + width] = mex * (1.0 / se)


@jax.jit
def kernel(x, importance, proj_all_W, proj_all_b, proj_fk_W, proj_fk_b,
           proj_rk_W, proj_rk_b, neuron_emb):
    del importance
    b, s, d = x.shape
    n_tok = b * s
    xf = x.reshape(n_tok, d)

    w_cat = jnp.concatenate([proj_all_W, proj_fk_W, proj_rk_W], axis=0)
    b_cat = jnp.concatenate([proj_all_b, proj_fk_b, proj_rk_b],
                            axis=0).reshape(1, N_PROJ)

    grid = (N_TILES + 1,)
    out = pl.pallas_call(
        _router_kernel,
        grid=grid,
        in_specs=[
            pl.BlockSpec((TILE, d), lambda i: (jnp.minimum(i, N_TILES - 1), 0)),
            pl.BlockSpec((N_PROJ, d), lambda i: (0, 0)),
            pl.BlockSpec((1, N_PROJ), lambda i: (0, 0)),
            pl.BlockSpec((3072, D_SPACE), lambda i: (0, 0)),
        ],
        out_specs=pl.BlockSpec(
            (TILE, N_OUT), lambda i: (jnp.maximum(i - 1, 0), 0)),
        out_shape=jax.ShapeDtypeStruct((n_tok, N_OUT), jnp.float32),
        scratch_shapes=[
            pltpu.VMEM((d, N_PROJ), jnp.bfloat16),
            pltpu.VMEM((N_PROJ, N_OUT), jnp.bfloat16),
            pltpu.VMEM((2, TILE, N_OUT), jnp.float32),
        ],
    )(xf, w_cat, b_cat, neuron_emb)
    return out.reshape(b, s, N_OUT)


# speedup vs baseline: 1.2029x; 1.2029x over previous
import jax
import jax.numpy as jnp
from jax.experimental import pallas as pl
from jax.experimental.pallas import tpu as pltpu

D_MODEL = 2048
D_SPACE = 64
GROUPS = (
    (0, 256, 8),
    (256, 256, 8),
    (512, 256, 8),
    (768, 256, 8),
    (1024, 256, 8),
    (1280, 1024, 4),
    (2304, 1024, 4),
)
EMB_ROWS = ((0, 256), (256, 512), (512, 768), (512, 768), (768, 1024),
            (1024, 2048), (2048, 3072))
N_OUT = 3328
N_PROJ = 448
TILE = 256


def _router_kernel(x_ref, w_ref, b_ref, emb_ref, o_ref, w_s, e5_s, e2_s):
    @pl.when(pl.program_id(0) == 0)
    def _prep():
        w_s[...] = jnp.zeros((D_MODEL, 512), dtype=jnp.bfloat16)
        w_s[:, 0:320] = w_ref[0:320].T.astype(jnp.bfloat16)
        w_s[:, 384:512] = w_ref[320:448].T.astype(jnp.bfloat16)
        emb = emb_ref[...]
        norm = jnp.maximum(
            jnp.sqrt(jnp.sum(emb * emb, axis=-1, keepdims=True)), 1e-12)
        emb_n = (emb / norm).astype(jnp.bfloat16)
        e5_s[...] = jnp.zeros((384, 1280), dtype=jnp.bfloat16)
        e2_s[...] = jnp.zeros((128, 2048), dtype=jnp.bfloat16)
        for g in range(5):
            a, bb = EMB_ROWS[g]
            e5_s[64 * g:64 * (g + 1), 256 * g:256 * (g + 1)] = emb_n[a:bb].T
        e2_s[0:64, 0:1024] = emb_n[1024:2048].T
        e2_s[64:128, 1024:2048] = emb_n[2048:3072].T

    xb = x_ref[...].astype(jnp.bfloat16)
    h = jnp.dot(xb, w_s[...], preferred_element_type=jnp.float32)
    h = (h + b_ref[...]).astype(jnp.bfloat16)
    logits5 = jnp.dot(h[:, 0:384], e5_s[...],
                      preferred_element_type=jnp.float32)
    logits2 = jnp.dot(h[:, 384:512], e2_s[...],
                      preferred_element_type=jnp.float32)

    for off, width, k in GROUPS:
        if width == 256:
            l = logits5[:, off:off + width]
        else:
            l = logits2[:, off - 1280:off - 1280 + width]
        m = jnp.max(l, axis=-1, keepdims=True)
        cur, mv = l, m
        for _ in range(k - 1):
            cur = jnp.where(cur >= mv, -jnp.inf, cur)
            mv = jnp.max(cur, axis=-1, keepdims=True)
        ex = jnp.exp(l - m)
        mex = jnp.where(l >= mv, ex, 0.0)
        se = jnp.sum(mex, axis=-1, keepdims=True)
        o_ref[:, off:off + width] = mex * (1.0 / se)


@jax.jit
def kernel(x, importance, proj_all_W, proj_all_b, proj_fk_W, proj_fk_b,
           proj_rk_W, proj_rk_b, neuron_emb):
    del importance
    b, s, d = x.shape
    n_tok = b * s
    xf = x.reshape(n_tok, d)

    w_cat = jnp.concatenate([proj_all_W, proj_fk_W, proj_rk_W], axis=0)
    b_cat = jnp.zeros((1, 512), dtype=jnp.float32)
    b_cat = b_cat.at[0, 0:320].set(proj_all_b)
    b_cat = b_cat.at[0, 384:448].set(proj_fk_b)
    b_cat = b_cat.at[0, 448:512].set(proj_rk_b)

    grid = (n_tok // TILE,)
    out = pl.pallas_call(
        _router_kernel,
        grid=grid,
        in_specs=[
            pl.BlockSpec((TILE, d), lambda i: (i, 0)),
            pl.BlockSpec((N_PROJ, d), lambda i: (0, 0)),
            pl.BlockSpec((1, 512), lambda i: (0, 0)),
            pl.BlockSpec((3072, D_SPACE), lambda i: (0, 0)),
        ],
        out_specs=pl.BlockSpec((TILE, N_OUT), lambda i: (i, 0)),
        out_shape=jax.ShapeDtypeStruct((n_tok, N_OUT), jnp.float32),
        scratch_shapes=[
            pltpu.VMEM((d, 512), jnp.bfloat16),
            pltpu.VMEM((384, 1280), jnp.bfloat16),
            pltpu.VMEM((128, 2048), jnp.bfloat16),
        ],
    )(xf, w_cat, b_cat, neuron_emb)
    return out.reshape(b, s, N_OUT)


# speedup vs baseline: 1.3428x; 1.1163x over previous
import jax
import jax.numpy as jnp
from jax.experimental import pallas as pl
from jax.experimental.pallas import tpu as pltpu

D_MODEL = 2048
D_SPACE = 64
GROUPS = (
    (0, 256, 8),
    (256, 256, 8),
    (512, 256, 8),
    (768, 256, 8),
    (1024, 256, 8),
    (1280, 1024, 4),
    (2304, 1024, 4),
)
EMB_ROWS = ((0, 256), (256, 512), (512, 768), (512, 768), (768, 1024),
            (1024, 2048), (2048, 3072))
N_OUT = 3328
N_PROJ = 448
TILE = 256


def _router_kernel(x_ref, w_ref, b_ref, emb_ref, o_ref, w_s, e_s):
    @pl.when(pl.program_id(0) == 0)
    def _prep():
        w_s[...] = w_ref[...].T.astype(jnp.bfloat16)
        emb = emb_ref[...]
        norm = jnp.maximum(
            jnp.sqrt(jnp.sum(emb * emb, axis=-1, keepdims=True)), 1e-12)
        emb_n = (emb / norm).astype(jnp.bfloat16)
        e_s[...] = jnp.zeros((N_PROJ, N_OUT), dtype=jnp.bfloat16)
        for g, (off, width, _) in enumerate(GROUPS):
            a, bb = EMB_ROWS[g]
            e_s[64 * g:64 * (g + 1), off:off + width] = emb_n[a:bb].T

    xb = x_ref[...].astype(jnp.bfloat16)
    h = jnp.dot(xb, w_s[...], preferred_element_type=jnp.float32)
    h = (h + b_ref[...]).astype(jnp.bfloat16)
    logits = jnp.dot(h, e_s[...], preferred_element_type=jnp.float32)

    for off, width, k in GROUPS:
        l = logits[:, off:off + width]
        m = jnp.max(l, axis=-1, keepdims=True)
        cur, mv = l, m
        for _ in range(k - 1):
            cur = jnp.where(cur >= mv, -jnp.inf, cur)
            mv = jnp.max(cur, axis=-1, keepdims=True)
        ex = jnp.exp(l - m)
        mex = jnp.where(l >= mv, ex, 0.0)
        se = jnp.sum(mex, axis=-1, keepdims=True)
        o_ref[:, off:off + width] = mex * (1.0 / se)


@jax.jit
def kernel(x, importance, proj_all_W, proj_all_b, proj_fk_W, proj_fk_b,
           proj_rk_W, proj_rk_b, neuron_emb):
    del importance
    b, s, d = x.shape
    n_tok = b * s
    xf = x.reshape(n_tok, d)

    w_cat = jnp.concatenate([proj_all_W, proj_fk_W, proj_rk_W], axis=0)
    b_cat = jnp.concatenate([proj_all_b, proj_fk_b, proj_rk_b],
                            axis=0).reshape(1, N_PROJ)

    grid = (n_tok // TILE,)
    out = pl.pallas_call(
        _router_kernel,
        grid=grid,
        in_specs=[
            pl.BlockSpec((TILE, d), lambda i: (i, 0)),
            pl.BlockSpec((N_PROJ, d), lambda i: (0, 0)),
            pl.BlockSpec((1, N_PROJ), lambda i: (0, 0)),
            pl.BlockSpec((3072, D_SPACE), lambda i: (0, 0)),
        ],
        out_specs=pl.BlockSpec((TILE, N_OUT), lambda i: (i, 0)),
        out_shape=jax.ShapeDtypeStruct((n_tok, N_OUT), jnp.float32),
        scratch_shapes=[
            pltpu.VMEM((d, N_PROJ), jnp.bfloat16),
            pltpu.VMEM((N_PROJ, N_OUT), jnp.bfloat16),
        ],
    )(xf, w_cat, b_cat, neuron_emb)
    return out.reshape(b, s, N_OUT)


# speedup vs baseline: 1.3457x; 1.0022x over previous
import jax
import jax.numpy as jnp
from jax.experimental import pallas as pl
from jax.experimental.pallas import tpu as pltpu

D_MODEL = 2048
D_SPACE = 64
GROUPS = (
    (0, 256, 8),
    (256, 256, 8),
    (512, 256, 8),
    (768, 256, 8),
    (1024, 256, 8),
    (1280, 1024, 4),
    (2304, 1024, 4),
)
EMB_ROWS = ((0, 256), (256, 512), (512, 768), (512, 768), (768, 1024),
            (1024, 2048), (2048, 3072))
N_OUT = 3328
N_PROJ = 448
TILE = 256


def _router_kernel(x_ref, w_ref, b_ref, emb_ref, o_ref, w_s, e_s):
    @pl.when(pl.program_id(0) == 0)
    def _prep():
        w_s[...] = w_ref[...].T.astype(jnp.bfloat16)
        emb = emb_ref[...]
        norm = jnp.maximum(
            jnp.sqrt(jnp.sum(emb * emb, axis=-1, keepdims=True)), 1e-12)
        emb_n = (emb / norm).astype(jnp.bfloat16)
        e_s[...] = jnp.zeros((N_PROJ, N_OUT), dtype=jnp.bfloat16)
        for g, (off, width, _) in enumerate(GROUPS):
            a, bb = EMB_ROWS[g]
            e_s[64 * g:64 * (g + 1), off:off + width] = emb_n[a:bb].T

    xb = x_ref[...].astype(jnp.bfloat16)
    h = jnp.dot(xb, w_s[...], preferred_element_type=jnp.float32)
    h = (h + b_ref[...]).astype(jnp.bfloat16)
    logits = jnp.dot(h, e_s[...], preferred_element_type=jnp.float32)

    for gi in (5, 0, 1, 6, 2, 3, 4):
        off, width, k = GROUPS[gi]
        l = logits[:, off:off + width]
        m = jnp.max(l, axis=-1, keepdims=True)
        cur, mv = l, m
        for _ in range(k - 1):
            cur = jnp.where(cur >= mv, -jnp.inf, cur)
            mv = jnp.max(cur, axis=-1, keepdims=True)
        ex = jnp.exp(l - m)
        mex = jnp.where(l >= mv, ex, 0.0)
        se = jnp.sum(mex, axis=-1, keepdims=True)
        o_ref[:, off:off + width] = mex * (1.0 / se)


@jax.jit
def kernel(x, importance, proj_all_W, proj_all_b, proj_fk_W, proj_fk_b,
           proj_rk_W, proj_rk_b, neuron_emb):
    del importance
    b, s, d = x.shape
    n_tok = b * s
    xf = x.reshape(n_tok, d)

    w_cat = jnp.concatenate([proj_all_W, proj_fk_W, proj_rk_W], axis=0)
    b_cat = jnp.concatenate([proj_all_b, proj_fk_b, proj_rk_b],
                            axis=0).reshape(1, N_PROJ)

    grid = (n_tok // TILE,)
    out = pl.pallas_call(
        _router_kernel,
        grid=grid,
        in_specs=[
            pl.BlockSpec((TILE, d), lambda i: (i, 0)),
            pl.BlockSpec((N_PROJ, d), lambda i: (0, 0)),
            pl.BlockSpec((1, N_PROJ), lambda i: (0, 0)),
            pl.BlockSpec((3072, D_SPACE), lambda i: (0, 0)),
        ],
        out_specs=pl.BlockSpec((TILE, N_OUT), lambda i: (i, 0)),
        out_shape=jax.ShapeDtypeStruct((n_tok, N_OUT), jnp.float32),
        scratch_shapes=[
            pltpu.VMEM((d, N_PROJ), jnp.bfloat16),
            pltpu.VMEM((N_PROJ, N_OUT), jnp.bfloat16),
        ],
    )(xf, w_cat, b_cat, neuron_emb)
    return out.reshape(b, s, N_OUT)
